# baseline (speedup 1.0000x reference)
import functools

import jax
import jax.numpy as jnp
from jax import lax
from jax.experimental import pallas as pl
from jax.experimental.pallas import tpu as pltpu
from jax.experimental.pallas import tpu_sc as plsc

_EPS = 1e-5
_EBLK = 1000
_NBLK = 1000
_CHUNK = 128
_NTILES = 32
_DEGW = 16


def _edge_body(dt_ref, h_ref, ef_ref, ew_ref, freq_ref,
               w1h_ref, w1e_ref, w1t_ref, b1_ref, w2_ref, b2_ref,
               w3_ref, b3_ref, out_ref):
    t = jnp.cos(dt_ref[...] * freq_ref[...])
    x = (jnp.dot(h_ref[...], w1h_ref[...], preferred_element_type=jnp.float32)
         + jnp.dot(ef_ref[...], w1e_ref[...], preferred_element_type=jnp.float32)
         + jnp.dot(t, w1t_ref[...], preferred_element_type=jnp.float32)
         + b1_ref[...])
    x = jnp.maximum(x, 0.0)
    x = jnp.maximum(
        jnp.dot(x, w2_ref[...], preferred_element_type=jnp.float32) + b2_ref[...], 0.0)
    out_ref[...] = (jnp.dot(x, w3_ref[...], preferred_element_type=jnp.float32)
                    + b3_ref[...]) * ew_ref[...]


def _edge_mlp(h, edge_f, dt, edge_w, time_freq,
              tp_W1, tp_b1, tp_W2, tp_b2, tp_W3, tp_b3, n_dst):
    e, d_node = edge_f.shape[0], h.shape[1]
    d_edge = edge_f.shape[1]
    d_time = time_freq.shape[0]
    d_out = tp_W3.shape[1]
    w1h = tp_W1[:d_node]
    w1e = tp_W1[d_node:d_node + d_edge]
    w1t = tp_W1[d_node + d_edge:]
    grid = e // _EBLK
    off = n_dst // _EBLK

    def full(a):
        return pl.BlockSpec(a.shape, lambda i: (0,) * a.ndim)

    return pl.pallas_call(
        _edge_body,
        grid=(grid,),
        in_specs=[
            pl.BlockSpec((_EBLK, 1), lambda i: (i, 0)),
            pl.BlockSpec((_EBLK, d_node), lambda i: (off + i, 0)),
            pl.BlockSpec((_EBLK, d_edge), lambda i: (i, 0)),
            pl.BlockSpec((_EBLK, 1), lambda i: (i, 0)),
            full(time_freq.reshape(1, d_time)),
            full(w1h), full(w1e), full(w1t),
            full(tp_b1.reshape(1, d_out)),
            full(tp_W2), full(tp_b2.reshape(1, d_out)),
            full(tp_W3), full(tp_b3.reshape(1, d_out)),
        ],
        out_specs=pl.BlockSpec((_EBLK, d_out), lambda i: (i, 0)),
        out_shape=jax.ShapeDtypeStruct((e, d_out), jnp.float32),
    )(dt.reshape(e, 1), h, edge_f, edge_w, time_freq.reshape(1, d_time),
      w1h, w1e, w1t, tp_b1.reshape(1, d_out),
      tp_W2, tp_b2.reshape(1, d_out), tp_W3, tp_b3.reshape(1, d_out))


_WIN = 320
_TRASH = 8


def _scatter_body(v_hbm, idx_hbm, bounds_hbm, h2_hbm, deg_hbm,
                  vbuf0, vbuf1, ibuf, bbuf, acc, dacc, semv0, semv1):
    c = lax.axis_index("c")
    s = lax.axis_index("s")
    t = c * 16 + s
    d = acc.shape[1]
    lo = t * _WIN
    vbufs = (vbuf0, vbuf1)
    semvs = (semv0, semv1)

    zeros16 = jnp.zeros((16,), jnp.float32)
    ones16 = jnp.ones((16,), jnp.float32)

    def zrow(i, _):
        def zcol(j, _):
            acc[i, pl.ds(j * 16, 16)] = zeros16
            return 0
        lax.fori_loop(0, d // 16, zcol, 0)
        dacc[i, pl.ds(0, 16)] = zeros16
        return 0
    lax.fori_loop(0, acc.shape[0], zrow, 0)

    pltpu.sync_copy(bounds_hbm, bbuf)
    brow = bbuf[t, pl.ds(0, 16)]
    c0 = brow[0]
    c1 = brow[1]
    last = c1 - 1

    def fire(j, b):
        pltpu.async_copy(v_hbm.at[pl.ds(j * _CHUNK, _CHUNK)], vbufs[b],
                         semvs[b])

    def wait(b):
        pltpu.make_async_copy(v_hbm.at[pl.ds(0, _CHUNK)], vbufs[b],
                              semvs[b]).wait()

    ncol = d // 16
    ngrp = _CHUNK // 16

    @pl.when(c1 > c0)
    def _():
        @pl.when(c0 % 2 == 0)
        def _():
            fire(c0, 0)

        @pl.when(c0 % 2 == 1)
        def _():
            fire(c0, 1)

        g0 = c0 // 8
        g1 = (c1 + 7) // 8

        def chunk_step(j, q, p):
            wait(p)
            fire(jnp.minimum(j + 1, last), 1 - p)
            vbuf = vbufs[p]

            def half(hh, _):
                for g2 in range(ngrp // 2):
                    iv = ibuf[q, pl.ds(hh * (_CHUNK // 2) + g2 * 16, 16)]
                    lv = iv - lo
                    inb = jnp.logical_and(lv >= 0, lv < _WIN)
                    rvec = jnp.where(inb, lv, _WIN)
                    for i in range(16):
                        r = rvec[i]
                        e = hh * (_CHUNK // 2) + g2 * 16 + i
                        for k in range(ncol):
                            acc[r, pl.ds(k * 16, 16)] = (
                                acc[r, pl.ds(k * 16, 16)]
                                + vbuf[e, pl.ds(k * 16, 16)])
                        dacc[r, pl.ds(0, 16)] = dacc[r, pl.ds(0, 16)] + ones16
                return 0
            lax.fori_loop(0, 2, half, 0)

        def group(gi, _):
            gg = g0 + gi
            pltpu.sync_copy(idx_hbm.at[pl.ds(gg * 8, 8)], ibuf)

            def pair(m, _):
                q0 = m * 2
                j0 = gg * 8 + q0

                @pl.when(jnp.logical_and(j0 >= c0, j0 < c1))
                def _():
                    chunk_step(j0, q0, 0)

                @pl.when(jnp.logical_and(j0 + 1 >= c0, j0 + 1 < c1))
                def _():
                    chunk_step(j0 + 1, q0 + 1, 1)
                return 0
            lax.fori_loop(0, 4, pair, 0)
            return 0
        lax.fori_loop(0, g1 - g0, group, 0)

        @pl.when(c1 % 2 == 0)
        def _():
            wait(0)

        @pl.when(c1 % 2 == 1)
        def _():
            wait(1)

    pltpu.sync_copy(acc.at[pl.ds(0, _WIN)], h2_hbm.at[pl.ds(lo, _WIN)])
    pltpu.sync_copy(dacc.at[pl.ds(0, _WIN)], deg_hbm.at[pl.ds(lo, _WIN)])


def _segment_scatter(v, dst_idx, n_dst):
    e, d = v.shape
    nchunks = e // _CHUNK
    idx32 = dst_idx.astype(jnp.int32)
    idx2d = idx32.reshape(nchunks, _CHUNK)
    npadrows = ((nchunks + 7) // 8) * 8 - nchunks
    if npadrows:
        idx2d = jnp.pad(idx2d, ((0, npadrows), (0, 0)))
    cut = jnp.searchsorted(idx32, jnp.arange(33, dtype=jnp.int32) * _WIN)
    c0 = (cut[:32] // _CHUNK).astype(jnp.int32)
    c1 = ((cut[1:] + _CHUNK - 1) // _CHUNK).astype(jnp.int32)
    bounds = jnp.zeros((32, 16), jnp.int32)
    bounds = bounds.at[:, 0].set(c0).at[:, 1].set(c1)
    mesh = plsc.VectorSubcoreMesh(core_axis_name="c", subcore_axis_name="s")
    fn = functools.partial(
        pl.kernel,
        out_type=[
            jax.ShapeDtypeStruct((32 * _WIN, d), jnp.float32),
            jax.ShapeDtypeStruct((32 * _WIN, _DEGW), jnp.float32),
        ],
        mesh=mesh,
        scratch_types=[
            pltpu.VMEM((_CHUNK, d), jnp.float32),
            pltpu.VMEM((_CHUNK, d), jnp.float32),
            pltpu.VMEM((8, _CHUNK), jnp.int32),
            pltpu.VMEM((32, 16), jnp.int32),
            pltpu.VMEM((_WIN + _TRASH, d), jnp.float32),
            pltpu.VMEM((_WIN + _TRASH, _DEGW), jnp.float32),
            pltpu.SemaphoreType.DMA,
            pltpu.SemaphoreType.DMA,
        ],
    )(_scatter_body)
    return fn(v, idx2d, bounds)


def _final_body(h2_ref, deg_ref, h_ref,
                w1_ref, b1_ref, w2_ref, b2_ref, w3_ref, b3_ref,
                lng_ref, lnb_ref, ln2g_ref, ln2b_ref, out_ref):
    h2 = h2_ref[...]
    deg = deg_ref[:, 0:1]
    h1 = h2 / jnp.maximum(deg, 1.0)
    x = (jnp.dot(h1, w1_ref[0], preferred_element_type=jnp.float32)
         + jnp.dot(h_ref[...], w1_ref[1], preferred_element_type=jnp.float32)
         + b1_ref[...])
    x = jnp.maximum(x, 0.0)
    x = jnp.maximum(
        jnp.dot(x, w2_ref[...], preferred_element_type=jnp.float32) + b2_ref[...], 0.0)
    rst = jnp.dot(x, w3_ref[...], preferred_element_type=jnp.float32) + b3_ref[...]

    def layer_norm(y, g, b):
        mu = jnp.mean(y, axis=-1, keepdims=True)
        var = jnp.mean(jnp.square(y - mu), axis=-1, keepdims=True)
        return (y - mu) * jax.lax.rsqrt(var + _EPS) * g + b

    out_ref[...] = (layer_norm(rst, lng_ref[...], lnb_ref[...])
                    + layer_norm(h2, ln2g_ref[...], ln2b_ref[...]))


def _finalize(h2, deg, h, cf_W1, cf_b1, cf_W2, cf_b2, cf_W3, cf_b3,
              ln_g, ln_b, ln2_g, ln2_b, n_dst):
    d = h2.shape[1]
    d_out = cf_W3.shape[1]
    grid = n_dst // _NBLK
    w1 = cf_W1.reshape(2, d, d_out)

    def full(a):
        return pl.BlockSpec(a.shape, lambda i: (0,) * a.ndim)

    return pl.pallas_call(
        _final_body,
        grid=(grid,),
        in_specs=[
            pl.BlockSpec((_NBLK, d), lambda i: (i, 0)),
            pl.BlockSpec((_NBLK, _DEGW), lambda i: (i, 0)),
            pl.BlockSpec((_NBLK, h.shape[1]), lambda i: (i, 0)),
            full(w1), full(cf_b1.reshape(1, d_out)),
            full(cf_W2), full(cf_b2.reshape(1, d_out)),
            full(cf_W3), full(cf_b3.reshape(1, d_out)),
            full(ln_g.reshape(1, d_out)), full(ln_b.reshape(1, d_out)),
            full(ln2_g.reshape(1, d_out)), full(ln2_b.reshape(1, d_out)),
        ],
        out_specs=pl.BlockSpec((_NBLK, d_out), lambda i: (i, 0)),
        out_shape=jax.ShapeDtypeStruct((n_dst, d_out), jnp.float32),
    )(h2, deg, h, w1, cf_b1.reshape(1, d_out),
      cf_W2, cf_b2.reshape(1, d_out), cf_W3, cf_b3.reshape(1, d_out),
      ln_g.reshape(1, d_out), ln_b.reshape(1, d_out),
      ln2_g.reshape(1, d_out), ln2_b.reshape(1, d_out))


def kernel(h, edge_f, dt, edge_w, dst_idx, time_freq,
           tp_W1, tp_b1, tp_W2, tp_b2, tp_W3, tp_b3,
           cf_W1, cf_b1, cf_W2, cf_b2, cf_W3, cf_b3,
           ln_g, ln_b, ln2_g, ln2_b):
    n_dst = h.shape[0] - dt.shape[0]
    v = _edge_mlp(h, edge_f, dt, edge_w, time_freq,
                  tp_W1, tp_b1, tp_W2, tp_b2, tp_W3, tp_b3, n_dst)
    h2, deg = _segment_scatter(v, dst_idx, n_dst)
    return _finalize(h2, deg, h, cf_W1, cf_b1, cf_W2, cf_b2, cf_W3, cf_b3,
                     ln_g, ln_b, ln2_g, ln2_b, n_dst)


# speedup vs baseline: 1.0532x; 1.0532x over previous
import functools

import jax
import jax.numpy as jnp
from jax import lax
from jax.experimental import pallas as pl
from jax.experimental.pallas import tpu as pltpu
from jax.experimental.pallas import tpu_sc as plsc

_EPS = 1e-5
_EBLK = 1000
_NBLK = 1000
_CHUNK = 80
_NTILES = 32
_DEGW = 16


def _edge_body(dt_ref, h_ref, ef_ref, ew_ref, freq_ref,
               w1h_ref, w1e_ref, w1t_ref, b1_ref, w2_ref, b2_ref,
               w3_ref, b3_ref, out_ref):
    t = jnp.cos(dt_ref[...] * freq_ref[...])
    x = (jnp.dot(h_ref[...], w1h_ref[...], preferred_element_type=jnp.float32)
         + jnp.dot(ef_ref[...], w1e_ref[...], preferred_element_type=jnp.float32)
         + jnp.dot(t, w1t_ref[...], preferred_element_type=jnp.float32)
         + b1_ref[...])
    x = jnp.maximum(x, 0.0)
    x = jnp.maximum(
        jnp.dot(x, w2_ref[...], preferred_element_type=jnp.float32) + b2_ref[...], 0.0)
    out_ref[...] = (jnp.dot(x, w3_ref[...], preferred_element_type=jnp.float32)
                    + b3_ref[...]) * ew_ref[...]


def _edge_mlp(h, edge_f, dt, edge_w, time_freq,
              tp_W1, tp_b1, tp_W2, tp_b2, tp_W3, tp_b3, n_dst):
    e, d_node = edge_f.shape[0], h.shape[1]
    d_edge = edge_f.shape[1]
    d_time = time_freq.shape[0]
    d_out = tp_W3.shape[1]
    w1h = tp_W1[:d_node]
    w1e = tp_W1[d_node:d_node + d_edge]
    w1t = tp_W1[d_node + d_edge:]
    grid = e // _EBLK
    off = n_dst // _EBLK

    def full(a):
        return pl.BlockSpec(a.shape, lambda i: (0,) * a.ndim)

    return pl.pallas_call(
        _edge_body,
        grid=(grid,),
        in_specs=[
            pl.BlockSpec((_EBLK, 1), lambda i: (i, 0)),
            pl.BlockSpec((_EBLK, d_node), lambda i: (off + i, 0)),
            pl.BlockSpec((_EBLK, d_edge), lambda i: (i, 0)),
            pl.BlockSpec((_EBLK, 1), lambda i: (i, 0)),
            full(time_freq.reshape(1, d_time)),
            full(w1h), full(w1e), full(w1t),
            full(tp_b1.reshape(1, d_out)),
            full(tp_W2), full(tp_b2.reshape(1, d_out)),
            full(tp_W3), full(tp_b3.reshape(1, d_out)),
        ],
        out_specs=pl.BlockSpec((_EBLK, d_out), lambda i: (i, 0)),
        out_shape=jax.ShapeDtypeStruct((e, d_out), jnp.float32),
    )(dt.reshape(e, 1), h, edge_f, edge_w, time_freq.reshape(1, d_time),
      w1h, w1e, w1t, tp_b1.reshape(1, d_out),
      tp_W2, tp_b2.reshape(1, d_out), tp_W3, tp_b3.reshape(1, d_out))


_WIN = 320
_TRASH = 8


def _scatter_body(v_hbm, idx_hbm, bounds_hbm, h2_hbm, deg_hbm,
                  vbuf0, vbuf1, ibuf0, ibuf1, bbuf, acc, dacc,
                  semv0, semv1, semi0, semi1):
    c = lax.axis_index("c")
    s = lax.axis_index("s")
    t = c * 16 + s
    d = acc.shape[1]
    lo = t * _WIN
    vbufs = (vbuf0, vbuf1)
    ibufs = (ibuf0, ibuf1)
    semvs = (semv0, semv1)
    semis = (semi0, semi1)

    zeros16 = jnp.zeros((16,), jnp.float32)
    ones16 = jnp.ones((16,), jnp.float32)

    def zrow(i, _):
        def zcol(j, _):
            acc[i, pl.ds(j * 16, 16)] = zeros16
            return 0
        lax.fori_loop(0, d // 16, zcol, 0)
        dacc[i, pl.ds(0, 16)] = zeros16
        return 0
    lax.fori_loop(0, acc.shape[0], zrow, 0)

    pltpu.sync_copy(bounds_hbm, bbuf)
    brow = bbuf[t, pl.ds(0, 16)]
    c0 = brow[0]
    c1 = brow[1]
    last = c1 - 1

    def fire(j, b):
        pltpu.async_copy(v_hbm.at[pl.ds(j * _CHUNK, _CHUNK)], vbufs[b],
                         semvs[b])
        pltpu.async_copy(idx_hbm.at[j], ibufs[b], semis[b])

    def wait(b):
        pltpu.make_async_copy(v_hbm.at[pl.ds(0, _CHUNK)], vbufs[b],
                              semvs[b]).wait()
        pltpu.make_async_copy(idx_hbm.at[0], ibufs[b], semis[b]).wait()

    ncol = d // 16

    @pl.when(c1 > c0)
    def _():
        fire(c0, 0)

        def step_b(j, b, vbuf, ibuf):
            wait(b)
            fire(jnp.minimum(j + 1, last), 1 - b)
            for g in range(_CHUNK // 16):
                iv = ibuf[0, pl.ds(g * 16, 16)]
                lv = iv - lo
                inb = jnp.logical_and(lv >= 0, lv < _WIN)
                rvec = jnp.where(inb, lv, _WIN)
                for i in range(16):
                    r = rvec[i]
                    e = g * 16 + i
                    for k in range(ncol):
                        acc[r, pl.ds(k * 16, 16)] = (
                            acc[r, pl.ds(k * 16, 16)]
                            + vbuf[e, pl.ds(k * 16, 16)])
                    dacc[r, pl.ds(0, 16)] = dacc[r, pl.ds(0, 16)] + ones16

        def pair(j2, _):
            j = c0 + j2 * 2

            @pl.when(j < c1)
            def _():
                step_b(j, 0, vbuf0, ibuf0)

            @pl.when(j + 1 < c1)
            def _():
                step_b(j + 1, 1, vbuf1, ibuf1)
            return 0
        lax.fori_loop(0, (c1 - c0 + 1) // 2, pair, 0)

        parity = (c1 - c0) % 2

        @pl.when(parity == 0)
        def _():
            wait(0)

        @pl.when(parity == 1)
        def _():
            wait(1)

    pltpu.sync_copy(acc.at[pl.ds(0, _WIN)], h2_hbm.at[pl.ds(lo, _WIN)])
    pltpu.sync_copy(dacc.at[pl.ds(0, _WIN)], deg_hbm.at[pl.ds(lo, _WIN)])


def _segment_scatter(v, dst_idx, n_dst):
    e, d = v.shape
    nchunks = e // _CHUNK
    idx32 = dst_idx.astype(jnp.int32)
    idx3d = idx32.reshape(nchunks, 1, _CHUNK)
    cut = jnp.searchsorted(idx32, jnp.arange(33, dtype=jnp.int32) * _WIN)
    c0 = (cut[:32] // _CHUNK).astype(jnp.int32)
    c1 = ((cut[1:] + _CHUNK - 1) // _CHUNK).astype(jnp.int32)
    bounds = jnp.zeros((32, 16), jnp.int32)
    bounds = bounds.at[:, 0].set(c0).at[:, 1].set(c1)
    mesh = plsc.VectorSubcoreMesh(core_axis_name="c", subcore_axis_name="s")
    fn = functools.partial(
        pl.kernel,
        out_type=[
            jax.ShapeDtypeStruct((32 * _WIN, d), jnp.float32),
            jax.ShapeDtypeStruct((32 * _WIN, _DEGW), jnp.float32),
        ],
        mesh=mesh,
        scratch_types=[
            pltpu.VMEM((_CHUNK, d), jnp.float32),
            pltpu.VMEM((_CHUNK, d), jnp.float32),
            pltpu.VMEM((1, _CHUNK), jnp.int32),
            pltpu.VMEM((1, _CHUNK), jnp.int32),
            pltpu.VMEM((32, 16), jnp.int32),
            pltpu.VMEM((_WIN + _TRASH, d), jnp.float32),
            pltpu.VMEM((_WIN + _TRASH, _DEGW), jnp.float32),
            pltpu.SemaphoreType.DMA,
            pltpu.SemaphoreType.DMA,
            pltpu.SemaphoreType.DMA,
            pltpu.SemaphoreType.DMA,
        ],
    )(_scatter_body)
    return fn(v, idx3d, bounds)


def _final_body(h2_ref, deg_ref, h_ref,
                w1_ref, b1_ref, w2_ref, b2_ref, w3_ref, b3_ref,
                lng_ref, lnb_ref, ln2g_ref, ln2b_ref, out_ref):
    h2 = h2_ref[...]
    deg = deg_ref[:, 0:1]
    h1 = h2 / jnp.maximum(deg, 1.0)
    x = (jnp.dot(h1, w1_ref[0], preferred_element_type=jnp.float32)
         + jnp.dot(h_ref[...], w1_ref[1], preferred_element_type=jnp.float32)
         + b1_ref[...])
    x = jnp.maximum(x, 0.0)
    x = jnp.maximum(
        jnp.dot(x, w2_ref[...], preferred_element_type=jnp.float32) + b2_ref[...], 0.0)
    rst = jnp.dot(x, w3_ref[...], preferred_element_type=jnp.float32) + b3_ref[...]

    def layer_norm(y, g, b):
        mu = jnp.mean(y, axis=-1, keepdims=True)
        var = jnp.mean(jnp.square(y - mu), axis=-1, keepdims=True)
        return (y - mu) * jax.lax.rsqrt(var + _EPS) * g + b

    out_ref[...] = (layer_norm(rst, lng_ref[...], lnb_ref[...])
                    + layer_norm(h2, ln2g_ref[...], ln2b_ref[...]))


def _finalize(h2, deg, h, cf_W1, cf_b1, cf_W2, cf_b2, cf_W3, cf_b3,
              ln_g, ln_b, ln2_g, ln2_b, n_dst):
    d = h2.shape[1]
    d_out = cf_W3.shape[1]
    grid = n_dst // _NBLK
    w1 = cf_W1.reshape(2, d, d_out)

    def full(a):
        return pl.BlockSpec(a.shape, lambda i: (0,) * a.ndim)

    return pl.pallas_call(
        _final_body,
        grid=(grid,),
        in_specs=[
            pl.BlockSpec((_NBLK, d), lambda i: (i, 0)),
            pl.BlockSpec((_NBLK, _DEGW), lambda i: (i, 0)),
            pl.BlockSpec((_NBLK, h.shape[1]), lambda i: (i, 0)),
            full(w1), full(cf_b1.reshape(1, d_out)),
            full(cf_W2), full(cf_b2.reshape(1, d_out)),
            full(cf_W3), full(cf_b3.reshape(1, d_out)),
            full(ln_g.reshape(1, d_out)), full(ln_b.reshape(1, d_out)),
            full(ln2_g.reshape(1, d_out)), full(ln2_b.reshape(1, d_out)),
        ],
        out_specs=pl.BlockSpec((_NBLK, d_out), lambda i: (i, 0)),
        out_shape=jax.ShapeDtypeStruct((n_dst, d_out), jnp.float32),
    )(h2, deg, h, w1, cf_b1.reshape(1, d_out),
      cf_W2, cf_b2.reshape(1, d_out), cf_W3, cf_b3.reshape(1, d_out),
      ln_g.reshape(1, d_out), ln_b.reshape(1, d_out),
      ln2_g.reshape(1, d_out), ln2_b.reshape(1, d_out))


def kernel(h, edge_f, dt, edge_w, dst_idx, time_freq,
           tp_W1, tp_b1, tp_W2, tp_b2, tp_W3, tp_b3,
           cf_W1, cf_b1, cf_W2, cf_b2, cf_W3, cf_b3,
           ln_g, ln_b, ln2_g, ln2_b):
    n_dst = h.shape[0] - dt.shape[0]
    v = _edge_mlp(h, edge_f, dt, edge_w, time_freq,
                  tp_W1, tp_b1, tp_W2, tp_b2, tp_W3, tp_b3, n_dst)
    h2, deg = _segment_scatter(v, dst_idx, n_dst)
    return _finalize(h2, deg, h, cf_W1, cf_b1, cf_W2, cf_b2, cf_W3, cf_b3,
                     ln_g, ln_b, ln2_g, ln2_b, n_dst)
